# Initial kernel scaffold; baseline (speedup 1.0000x reference)
import jax
import jax.numpy as jnp
from jax.experimental import pallas as pl


def kernel(x, pos, token_embed, pos_embed):
    raise NotImplementedError("write your pallas kernel here")


# speedup vs baseline: 2.4773x; 2.4773x over previous
import functools

import jax
import jax.numpy as jnp
from jax import lax
from jax.experimental import pallas as pl
from jax.experimental.pallas import tpu as pltpu
from jax.experimental.pallas import tpu_sc as plsc

D = 128
NW = 32
TOTAL = 4096 * 50
R = TOTAL // NW
C = 128
NCH = R // C

_mesh = plsc.VectorSubcoreMesh(core_axis_name="c", subcore_axis_name="s")


@functools.partial(
    pl.kernel,
    out_type=jax.ShapeDtypeStruct((NW, NCH, C, D), jnp.float32),
    mesh=_mesh,
    scratch_types=[
        pltpu.VMEM((NCH, C), jnp.int32),
        pltpu.VMEM((NCH, C), jnp.int32),
        pltpu.VMEM((C, D), jnp.float32),
        pltpu.VMEM((C, D), jnp.float32),
        pltpu.SemaphoreType.DMA,
        pltpu.SemaphoreType.DMA,
        pltpu.SemaphoreType.DMA,
    ],
)
def _emb_lookup(x_hbm, pos_hbm, tok_tab_hbm, pos_tab_hbm, out_hbm,
                tok_idx, pos_idx, buf_tok, buf_pos,
                sem_tok, sem_pos, sem_out):
    w = lax.axis_index("s") * 2 + lax.axis_index("c")
    pltpu.sync_copy(x_hbm.at[w], tok_idx)
    pltpu.sync_copy(pos_hbm.at[w], pos_idx)

    def chunk(c, carry):
        cp_t = pltpu.async_copy(tok_tab_hbm.at[tok_idx.at[c]], buf_tok, sem_tok)
        cp_p = pltpu.async_copy(pos_tab_hbm.at[pos_idx.at[c]], buf_pos, sem_pos)
        cp_t.wait()
        cp_p.wait()

        def add_row(i, carry2):
            for j in range(D // 16):
                sl = pl.ds(j * 16, 16)
                plsc.addupdate(buf_tok.at[i, sl], buf_pos[i, sl])
            return carry2

        lax.fori_loop(0, C, add_row, 0)

        pltpu.async_copy(buf_tok, out_hbm.at[w, c], sem_out).wait()
        return carry

    lax.fori_loop(0, NCH, chunk, 0)


def kernel(x, pos, token_embed, pos_embed):
    xw = x.reshape(NW, NCH, C)
    pw = pos.reshape(NW, NCH, C)
    out = _emb_lookup(xw, pw, token_embed, pos_embed)
    return out.reshape(x.shape[0], x.shape[1], D)
